# Initial kernel scaffold; baseline (speedup 1.0000x reference)
import jax
import jax.numpy as jnp
from jax.experimental import pallas as pl


def kernel(x, adj, W1, b1, W2, b2):
    raise NotImplementedError("write your pallas kernel here")


# speedup vs baseline: 1.0234x; 1.0234x over previous
import functools

import jax
import jax.numpy as jnp
from jax.experimental import pallas as pl
from jax.experimental.pallas import tpu as pltpu


def _pick_bm(n, target=400):
    best = 1
    for bm in range(1, min(n, target) + 1):
        if n % bm == 0:
            if bm % 8 == 0 or best % 8 != 0:
                if bm > best or (bm % 8 == 0 and best % 8 != 0):
                    best = bm
    return best


def _l1_kernel(x_ref, adj_ref, w1_ref, b1_ref, w2_ref, s2_ref, s1_scr):
    i = pl.program_id(0)

    @pl.when(i == 0)
    def _():
        s1_scr[:] = jnp.dot(x_ref[:], w1_ref[:],
                            preferred_element_type=jnp.float32)

    h = jnp.dot(adj_ref[:], s1_scr[:], preferred_element_type=jnp.float32)
    h = jnp.maximum(h + b1_ref[:], 0.0)
    s2_ref[:] = jnp.dot(h, w2_ref[:], preferred_element_type=jnp.float32)


def _l2_kernel(s2_ref, adj_ref, b2_ref, out_ref):
    o = jnp.dot(adj_ref[:], s2_ref[:], preferred_element_type=jnp.float32)
    out_ref[:] = jax.nn.sigmoid(o + b2_ref[:])


@jax.jit
def kernel(x, adj, W1, b1, W2, b2):
    n, f = x.shape
    h_dim = W1.shape[1]
    l_dim = W2.shape[1]
    b1r = b1.reshape(1, h_dim)
    b2r = b2.reshape(1, l_dim)

    bm = _pick_bm(n)
    nm = n // bm
    params = pltpu.CompilerParams(
        dimension_semantics=("arbitrary",),
        vmem_limit_bytes=64 * 1024 * 1024,
    )

    s2 = pl.pallas_call(
        _l1_kernel,
        grid=(nm,),
        in_specs=[
            pl.BlockSpec((n, f), lambda i: (0, 0)),
            pl.BlockSpec((bm, n), lambda i: (i, 0)),
            pl.BlockSpec((f, h_dim), lambda i: (0, 0)),
            pl.BlockSpec((1, h_dim), lambda i: (0, 0)),
            pl.BlockSpec((h_dim, l_dim), lambda i: (0, 0)),
        ],
        out_specs=pl.BlockSpec((bm, l_dim), lambda i: (i, 0)),
        out_shape=jax.ShapeDtypeStruct((n, l_dim), jnp.float32),
        scratch_shapes=[pltpu.VMEM((n, h_dim), jnp.float32)],
        compiler_params=params,
    )(x, adj, W1, b1r, W2)

    out = pl.pallas_call(
        _l2_kernel,
        grid=(nm,),
        in_specs=[
            pl.BlockSpec((n, l_dim), lambda i: (0, 0)),
            pl.BlockSpec((bm, n), lambda i: (i, 0)),
            pl.BlockSpec((1, l_dim), lambda i: (0, 0)),
        ],
        out_specs=pl.BlockSpec((bm, l_dim), lambda i: (i, 0)),
        out_shape=jax.ShapeDtypeStruct((n, l_dim), jnp.float32),
        compiler_params=params,
    )(s2, adj, b2r)

    return out


# speedup vs baseline: 1.0531x; 1.0290x over previous
import jax
import jax.numpy as jnp
from jax.experimental import pallas as pl
from jax.experimental.pallas import tpu as pltpu


def _pick_bm(n, target=400):
    best = 1
    for bm in range(1, min(n, target) + 1):
        if n % bm == 0:
            if bm % 8 == 0 or best % 8 != 0:
                if bm > best or (bm % 8 == 0 and best % 8 != 0):
                    best = bm
    return best


def _gcn_kernel(x_ref, adj_ref, w1_ref, b1_ref, w2_ref, b2_ref, out_ref,
                s1_scr, s2_scr, *, bm):
    p = pl.program_id(0)
    i = pl.program_id(1)

    @pl.when((p == 0) & (i == 0))
    def _():
        s1_scr[:] = jnp.dot(x_ref[:], w1_ref[:],
                            preferred_element_type=jnp.float32)

    @pl.when(p == 0)
    def _():
        h = jnp.dot(adj_ref[:], s1_scr[:],
                    preferred_element_type=jnp.float32)
        h = jnp.maximum(h + b1_ref[:], 0.0)
        s2_scr[pl.ds(i * bm, bm), :] = jnp.dot(
            h, w2_ref[:], preferred_element_type=jnp.float32)

    @pl.when(p == 1)
    def _():
        o = jnp.dot(adj_ref[:], s2_scr[:],
                    preferred_element_type=jnp.float32)
        out_ref[:] = jax.nn.sigmoid(o + b2_ref[:])


@jax.jit
def kernel(x, adj, W1, b1, W2, b2):
    n, f = x.shape
    h_dim = W1.shape[1]
    l_dim = W2.shape[1]
    b1r = b1.reshape(1, h_dim)
    b2r = b2.reshape(1, l_dim)

    bm = _pick_bm(n)
    nm = n // bm
    import functools
    body = functools.partial(_gcn_kernel, bm=bm)

    out = pl.pallas_call(
        body,
        grid=(2, nm),
        in_specs=[
            pl.BlockSpec((n, f), lambda p, i: (0, 0)),
            pl.BlockSpec((bm, n), lambda p, i: (i, 0)),
            pl.BlockSpec((f, h_dim), lambda p, i: (0, 0)),
            pl.BlockSpec((1, h_dim), lambda p, i: (0, 0)),
            pl.BlockSpec((h_dim, l_dim), lambda p, i: (0, 0)),
            pl.BlockSpec((1, l_dim), lambda p, i: (0, 0)),
        ],
        out_specs=pl.BlockSpec((bm, l_dim), lambda p, i: (i, 0)),
        out_shape=jax.ShapeDtypeStruct((n, l_dim), jnp.float32),
        scratch_shapes=[
            pltpu.VMEM((n, h_dim), jnp.float32),
            pltpu.VMEM((n, l_dim), jnp.float32),
        ],
        compiler_params=pltpu.CompilerParams(
            dimension_semantics=("arbitrary", "arbitrary"),
            vmem_limit_bytes=64 * 1024 * 1024,
        ),
    )(x, adj, W1, b1r, W2, b2r)

    return out


# speedup vs baseline: 1.0545x; 1.0014x over previous
import jax
import jax.numpy as jnp
from jax.experimental import pallas as pl
from jax.experimental.pallas import tpu as pltpu


def _pick_bm(n, target=400):
    best = 1
    for bm in range(1, min(n, target) + 1):
        if n % bm == 0:
            if bm % 8 == 0 or best % 8 != 0:
                if bm > best or (bm % 8 == 0 and best % 8 != 0):
                    best = bm
    return best


def _gcn_kernel(x_ref, adj_ref, w1_ref, b1_ref, w2_ref, b2_ref, out_ref,
                s1_scr, s2_scr, *, bm):
    p = pl.program_id(0)
    i = pl.program_id(1)

    @pl.when((p == 0) & (i == 0))
    def _():
        s1_scr[:] = jnp.dot(x_ref[:], w1_ref[:],
                            preferred_element_type=jnp.float32)

    @pl.when(p == 0)
    def _():
        a16 = adj_ref[:].astype(jnp.bfloat16)
        h = jnp.dot(a16, s1_scr[:].astype(jnp.bfloat16),
                    preferred_element_type=jnp.float32)
        h = jnp.maximum(h + b1_ref[:], 0.0)
        s2_scr[pl.ds(i * bm, bm), :] = jnp.dot(
            h, w2_ref[:], preferred_element_type=jnp.float32)

    @pl.when(p == 1)
    def _():
        a16 = adj_ref[:].astype(jnp.bfloat16)
        o = jnp.dot(a16, s2_scr[:].astype(jnp.bfloat16),
                    preferred_element_type=jnp.float32)
        out_ref[:] = jax.nn.sigmoid(o + b2_ref[:])


@jax.jit
def kernel(x, adj, W1, b1, W2, b2):
    n, f = x.shape
    h_dim = W1.shape[1]
    l_dim = W2.shape[1]
    b1r = b1.reshape(1, h_dim)
    b2r = b2.reshape(1, l_dim)

    bm = _pick_bm(n)
    nm = n // bm
    import functools
    body = functools.partial(_gcn_kernel, bm=bm)

    out = pl.pallas_call(
        body,
        grid=(2, nm),
        in_specs=[
            pl.BlockSpec((n, f), lambda p, i: (0, 0)),
            pl.BlockSpec((bm, n), lambda p, i: (i, 0)),
            pl.BlockSpec((f, h_dim), lambda p, i: (0, 0)),
            pl.BlockSpec((1, h_dim), lambda p, i: (0, 0)),
            pl.BlockSpec((h_dim, l_dim), lambda p, i: (0, 0)),
            pl.BlockSpec((1, l_dim), lambda p, i: (0, 0)),
        ],
        out_specs=pl.BlockSpec((bm, l_dim), lambda p, i: (i, 0)),
        out_shape=jax.ShapeDtypeStruct((n, l_dim), jnp.float32),
        scratch_shapes=[
            pltpu.VMEM((n, h_dim), jnp.float32),
            pltpu.VMEM((n, l_dim), jnp.float32),
        ],
        compiler_params=pltpu.CompilerParams(
            dimension_semantics=("arbitrary", "arbitrary"),
            vmem_limit_bytes=64 * 1024 * 1024,
        ),
    )(x, adj, W1, b1r, W2, b2r)

    return out


# speedup vs baseline: 1.0552x; 1.0006x over previous
import jax
import jax.numpy as jnp
from jax.experimental import pallas as pl
from jax.experimental.pallas import tpu as pltpu


def _pick_bm(n, target=500):
    best = 1
    for bm in range(1, min(n, target) + 1):
        if n % bm == 0:
            if bm % 8 == 0 or best % 8 != 0:
                if bm > best or (bm % 8 == 0 and best % 8 != 0):
                    best = bm
    return best


def _gcn_kernel(x_ref, adj_ref, w1_ref, b1_ref, w2_ref, b2_ref, out_ref,
                s1_scr, s2_scr, *, bm):
    p = pl.program_id(0)
    i = pl.program_id(1)

    @pl.when((p == 0) & (i == 0))
    def _():
        s1_scr[:] = jnp.dot(x_ref[:], w1_ref[:],
                            preferred_element_type=jnp.float32)

    @pl.when(p == 0)
    def _():
        h = jnp.dot(adj_ref[:], s1_scr[:],
                    preferred_element_type=jnp.float32)
        h = jnp.maximum(h + b1_ref[:], 0.0)
        s2_scr[pl.ds(i * bm, bm), :] = jnp.dot(
            h, w2_ref[:], preferred_element_type=jnp.float32)

    @pl.when(p == 1)
    def _():
        o = jnp.dot(adj_ref[:], s2_scr[:],
                    preferred_element_type=jnp.float32)
        out_ref[:] = jax.nn.sigmoid(o + b2_ref[:])


@jax.jit
def kernel(x, adj, W1, b1, W2, b2):
    n, f = x.shape
    h_dim = W1.shape[1]
    l_dim = W2.shape[1]
    b1r = b1.reshape(1, h_dim)
    b2r = b2.reshape(1, l_dim)

    bm = _pick_bm(n)
    nm = n // bm
    import functools
    body = functools.partial(_gcn_kernel, bm=bm)

    out = pl.pallas_call(
        body,
        grid=(2, nm),
        in_specs=[
            pl.BlockSpec((n, f), lambda p, i: (0, 0)),
            pl.BlockSpec((bm, n), lambda p, i: (i, 0)),
            pl.BlockSpec((f, h_dim), lambda p, i: (0, 0)),
            pl.BlockSpec((1, h_dim), lambda p, i: (0, 0)),
            pl.BlockSpec((h_dim, l_dim), lambda p, i: (0, 0)),
            pl.BlockSpec((1, l_dim), lambda p, i: (0, 0)),
        ],
        out_specs=pl.BlockSpec(
            (bm, l_dim), lambda p, i: (jnp.where(p == 0, 0, i), 0)),
        out_shape=jax.ShapeDtypeStruct((n, l_dim), jnp.float32),
        scratch_shapes=[
            pltpu.VMEM((n, h_dim), jnp.float32),
            pltpu.VMEM((n, l_dim), jnp.float32),
        ],
        compiler_params=pltpu.CompilerParams(
            dimension_semantics=("arbitrary", "arbitrary"),
            vmem_limit_bytes=64 * 1024 * 1024,
        ),
    )(x, adj, W1, b1r, W2, b2r)

    return out
